# baseline (speedup 1.0000x reference)
import functools

import jax
import jax.numpy as jnp
from jax import lax
from jax.experimental import pallas as pl
from jax.experimental.pallas import tpu as pltpu
from jax.experimental.pallas import tpu_sc as plsc


@functools.lru_cache(maxsize=None)
def _build(B, D, V_u, V_i):
    info = plsc.get_sparse_core_info()
    NC, NS, L = info.num_cores, info.num_subcores, info.num_lanes
    NW = NC * NS
    BPW = B // NW
    assert V_u == V_i

    mesh = plsc.VectorSubcoreMesh(core_axis_name="c", subcore_axis_name="s")

    @functools.partial(
        pl.kernel,
        mesh=mesh,
        compiler_params=pltpu.CompilerParams(needs_layout_passes=False),
        out_type=(
            jax.ShapeDtypeStruct((B,), jnp.float32),
            jax.ShapeDtypeStruct((B,), jnp.float32),
        ),
        scratch_types=[
            pltpu.VMEM((BPW,), jnp.int32),
            pltpu.VMEM((BPW,), jnp.int32),
            pltpu.VMEM((BPW,), jnp.int32),
            pltpu.VMEM((D,), jnp.int32),
            pltpu.VMEM((16, D, 128), jnp.float32),
            pltpu.VMEM((16, D, 128), jnp.float32),
            pltpu.VMEM((16, D, 128), jnp.float32),
            pltpu.VMEM((BPW,), jnp.float32),
            pltpu.VMEM((BPW,), jnp.float32),
        ] + [pltpu.SemaphoreType.DMA] * 16,
    )
    def bpr(u_hbm, i_hbm, j_hbm, ut_hbm, it_hbm, pos_hbm, neg_hbm,
            uv, iv, jv, dlist, uwin, iwin, jwin, posv, negv,
            *sems):
        wid = lax.axis_index("s") * NC + lax.axis_index("c")
        base = wid * BPW
        sl = pl.ds(base, BPW)
        pltpu.sync_copy(u_hbm.at[sl], uv)
        pltpu.sync_copy(i_hbm.at[sl], iv)
        pltpu.sync_copy(j_hbm.at[sl], jv)
        dlist[pl.ds(0, D)] = lax.iota(jnp.int32, D)

        NBUF = 16
        lanes = lax.iota(jnp.int32, L)
        lane0 = lanes == 0
        dreg = lax.iota(jnp.int32, D)

        def win(r):
            return pl.multiple_of((r >> 7) * 128, 128)

        def fire(ru, ri, rj, b):
            sem = sems[b]
            pltpu.async_copy(
                ut_hbm.at[:, pl.ds(win(ru), 128)], uwin.at[b], sem)
            pltpu.async_copy(
                it_hbm.at[:, pl.ds(win(ri), 128)], iwin.at[b], sem)
            pltpu.async_copy(
                it_hbm.at[:, pl.ds(win(rj), 128)], jwin.at[b], sem)

        def wait(b):
            sem = sems[b]
            pltpu.make_async_copy(
                ut_hbm.at[:, pl.ds(0, 128)], uwin.at[b], sem).wait()
            pltpu.make_async_copy(
                it_hbm.at[:, pl.ds(0, 128)], iwin.at[b], sem).wait()
            pltpu.make_async_copy(
                it_hbm.at[:, pl.ds(0, 128)], jwin.at[b], sem).wait()

        def compute(e, ru, ri, rj, b):
            urow = plsc.load_gather(
                uwin.at[b], [dreg, jnp.full((L,), ru & 127, jnp.int32)])
            irow = plsc.load_gather(
                iwin.at[b], [dreg, jnp.full((L,), ri & 127, jnp.int32)])
            jrow = plsc.load_gather(
                jwin.at[b], [dreg, jnp.full((L,), rj & 127, jnp.int32)])
            pos_e = jnp.sum(urow * irow)
            neg_e = jnp.sum(urow * jrow)
            ev = jnp.full((L,), e, jnp.int32)
            plsc.store_scatter(posv, [ev], jnp.full((L,), pos_e), mask=lane0)
            plsc.store_scatter(negv, [ev], jnp.full((L,), neg_e), mask=lane0)

        def block(k, carry):
            s = pl.ds(k * L, L)
            su = uv[s]
            si = iv[s]
            sj = jv[s]
            ru = [su[t] for t in range(L)]
            ri = [si[t] for t in range(L)]
            rj = [sj[t] for t in range(L)]
            for t in range(NBUF - 1):
                fire(ru[t], ri[t], rj[t], t)
            for t in range(L):
                if t + NBUF - 1 < L:
                    fire(ru[t + NBUF - 1], ri[t + NBUF - 1],
                         rj[t + NBUF - 1], (t + NBUF - 1) % NBUF)
                wait(t % NBUF)
                compute(k * L + t, ru[t], ri[t], rj[t], t % NBUF)
            return carry

        lax.fori_loop(0, BPW // L, block, 0)

        pltpu.sync_copy(posv, pos_hbm.at[sl])
        pltpu.sync_copy(negv, neg_hbm.at[sl])

    def run(u, i, j, user_table, item_table):
        return bpr(u, i, j, user_table.T, item_table.T)

    return run


def kernel(u, i, j, user_table, item_table):
    B = u.shape[0]
    D = user_table.shape[1]
    run = _build(B, D, user_table.shape[0], item_table.shape[0])
    return run(u, i, j, user_table, item_table)


# speedup vs baseline: 1.0125x; 1.0125x over previous
import functools

import jax
import jax.numpy as jnp
from jax import lax
from jax.experimental import pallas as pl
from jax.experimental.pallas import tpu as pltpu
from jax.experimental.pallas import tpu_sc as plsc


@functools.lru_cache(maxsize=None)
def _build(B, D, V_u, V_i):
    info = plsc.get_sparse_core_info()
    NC, NS, L = info.num_cores, info.num_subcores, info.num_lanes
    NW = NC * NS
    BPW = B // NW
    assert V_u == V_i

    mesh = plsc.VectorSubcoreMesh(core_axis_name="c", subcore_axis_name="s")

    @functools.partial(
        pl.kernel,
        mesh=mesh,
        compiler_params=pltpu.CompilerParams(needs_layout_passes=False),
        out_type=(
            jax.ShapeDtypeStruct((B,), jnp.float32),
            jax.ShapeDtypeStruct((B,), jnp.float32),
        ),
        scratch_types=[
            pltpu.VMEM((BPW,), jnp.int32),
            pltpu.VMEM((BPW,), jnp.int32),
            pltpu.VMEM((BPW,), jnp.int32),
            pltpu.VMEM((D,), jnp.int32),
            pltpu.VMEM((8, D, 128), jnp.float32),
            pltpu.VMEM((8, D, 128), jnp.float32),
            pltpu.VMEM((8, D, 128), jnp.float32),
            pltpu.VMEM((BPW,), jnp.float32),
            pltpu.VMEM((BPW,), jnp.float32),
        ] + [pltpu.SemaphoreType.DMA] * 8,
    )
    def bpr(u_hbm, i_hbm, j_hbm, ut_hbm, it_hbm, pos_hbm, neg_hbm,
            uv, iv, jv, dlist, uwin, iwin, jwin, posv, negv,
            *sems):
        wid = lax.axis_index("s") * NC + lax.axis_index("c")
        base = wid * BPW
        sl = pl.ds(base, BPW)
        pltpu.sync_copy(u_hbm.at[sl], uv)
        pltpu.sync_copy(i_hbm.at[sl], iv)
        pltpu.sync_copy(j_hbm.at[sl], jv)
        dlist[pl.ds(0, D)] = lax.iota(jnp.int32, D)

        NBUF = 8
        lanes = lax.iota(jnp.int32, L)
        lane0 = lanes == 0
        dreg = lax.iota(jnp.int32, D)

        def win(r):
            return pl.multiple_of((r >> 7) * 128, 128)

        def fire(ru, ri, rj, b):
            sem = sems[b]
            pltpu.async_copy(
                ut_hbm.at[:, pl.ds(win(ru), 128)], uwin.at[b], sem)
            pltpu.async_copy(
                it_hbm.at[:, pl.ds(win(ri), 128)], iwin.at[b], sem)
            pltpu.async_copy(
                it_hbm.at[:, pl.ds(win(rj), 128)], jwin.at[b], sem)

        def wait(b):
            sem = sems[b]
            pltpu.make_async_copy(
                ut_hbm.at[:, pl.ds(0, 128)], uwin.at[b], sem).wait()
            pltpu.make_async_copy(
                it_hbm.at[:, pl.ds(0, 128)], iwin.at[b], sem).wait()
            pltpu.make_async_copy(
                it_hbm.at[:, pl.ds(0, 128)], jwin.at[b], sem).wait()

        def compute(e, ru, ri, rj, b):
            urow = plsc.load_gather(
                uwin.at[b], [dreg, jnp.full((L,), ru & 127, jnp.int32)])
            irow = plsc.load_gather(
                iwin.at[b], [dreg, jnp.full((L,), ri & 127, jnp.int32)])
            jrow = plsc.load_gather(
                jwin.at[b], [dreg, jnp.full((L,), rj & 127, jnp.int32)])
            pos_e = jnp.sum(urow * irow)
            neg_e = jnp.sum(urow * jrow)
            ev = jnp.full((L,), e, jnp.int32)
            plsc.store_scatter(posv, [ev], jnp.full((L,), pos_e), mask=lane0)
            plsc.store_scatter(negv, [ev], jnp.full((L,), neg_e), mask=lane0)

        def block(k, carry):
            s = pl.ds(k * L, L)
            su = uv[s]
            si = iv[s]
            sj = jv[s]
            ru = [su[t] for t in range(L)]
            ri = [si[t] for t in range(L)]
            rj = [sj[t] for t in range(L)]
            for t in range(NBUF - 1):
                fire(ru[t], ri[t], rj[t], t)
            for t in range(L):
                if t + NBUF - 1 < L:
                    fire(ru[t + NBUF - 1], ri[t + NBUF - 1],
                         rj[t + NBUF - 1], (t + NBUF - 1) % NBUF)
                wait(t % NBUF)
                compute(k * L + t, ru[t], ri[t], rj[t], t % NBUF)
            return carry

        lax.fori_loop(0, BPW // L, block, 0)

        pltpu.sync_copy(posv, pos_hbm.at[sl])
        pltpu.sync_copy(negv, neg_hbm.at[sl])

    def run(u, i, j, user_table, item_table):
        return bpr(u, i, j, user_table.T, item_table.T)

    return run


def kernel(u, i, j, user_table, item_table):
    B = u.shape[0]
    D = user_table.shape[1]
    run = _build(B, D, user_table.shape[0], item_table.shape[0])
    return run(u, i, j, user_table, item_table)


# speedup vs baseline: 1.0127x; 1.0002x over previous
import functools

import jax
import jax.numpy as jnp
from jax import lax
from jax.experimental import pallas as pl
from jax.experimental.pallas import tpu as pltpu
from jax.experimental.pallas import tpu_sc as plsc


@functools.lru_cache(maxsize=None)
def _build(B, D, V_u, V_i):
    info = plsc.get_sparse_core_info()
    NC, NS, L = info.num_cores, info.num_subcores, info.num_lanes
    NW = NC * NS
    BPW = B // NW
    assert V_u == V_i

    mesh = plsc.VectorSubcoreMesh(core_axis_name="c", subcore_axis_name="s")

    @functools.partial(
        pl.kernel,
        mesh=mesh,
        compiler_params=pltpu.CompilerParams(needs_layout_passes=False),
        out_type=(
            jax.ShapeDtypeStruct((B,), jnp.float32),
            jax.ShapeDtypeStruct((B,), jnp.float32),
        ),
        scratch_types=[
            pltpu.VMEM((BPW,), jnp.int32),
            pltpu.VMEM((BPW,), jnp.int32),
            pltpu.VMEM((BPW,), jnp.int32),
            pltpu.VMEM((8, D, 128), jnp.float32),
            pltpu.VMEM((8, D, 128), jnp.float32),
            pltpu.VMEM((8, D, 128), jnp.float32),
            pltpu.VMEM((BPW,), jnp.float32),
            pltpu.VMEM((BPW,), jnp.float32),
        ] + [pltpu.SemaphoreType.DMA] * 8,
    )
    def bpr(u_hbm, i_hbm, j_hbm, ut_hbm, it_hbm, pos_hbm, neg_hbm,
            uv, iv, jv, uwin, iwin, jwin, posv, negv,
            *sems):
        wid = lax.axis_index("s") * NC + lax.axis_index("c")
        base = wid * BPW
        sl = pl.ds(base, BPW)
        pltpu.sync_copy(u_hbm.at[sl], uv)
        pltpu.sync_copy(i_hbm.at[sl], iv)
        pltpu.sync_copy(j_hbm.at[sl], jv)

        NBUF = 8
        lanes = lax.iota(jnp.int32, L)
        lane0 = lanes == 0
        dreg = lax.iota(jnp.int32, D)

        def win(r):
            return pl.multiple_of((r >> 7) * 128, 128)

        def fire(ru, ri, rj, b):
            sem = sems[b]
            pltpu.async_copy(
                ut_hbm.at[:, pl.ds(win(ru), 128)], uwin.at[b], sem)
            pltpu.async_copy(
                it_hbm.at[:, pl.ds(win(ri), 128)], iwin.at[b], sem)
            pltpu.async_copy(
                it_hbm.at[:, pl.ds(win(rj), 128)], jwin.at[b], sem)

        def wait(b):
            sem = sems[b]
            pltpu.make_async_copy(
                ut_hbm.at[:, pl.ds(0, 128)], uwin.at[b], sem).wait()
            pltpu.make_async_copy(
                it_hbm.at[:, pl.ds(0, 128)], iwin.at[b], sem).wait()
            pltpu.make_async_copy(
                it_hbm.at[:, pl.ds(0, 128)], jwin.at[b], sem).wait()

        def compute(e, ru, ri, rj, b):
            urow = plsc.load_gather(
                uwin.at[b], [dreg, jnp.full((L,), ru & 127, jnp.int32)])
            irow = plsc.load_gather(
                iwin.at[b], [dreg, jnp.full((L,), ri & 127, jnp.int32)])
            jrow = plsc.load_gather(
                jwin.at[b], [dreg, jnp.full((L,), rj & 127, jnp.int32)])
            pos_e = jnp.sum(urow * irow)
            neg_e = jnp.sum(urow * jrow)
            ev = jnp.full((L,), e, jnp.int32)
            plsc.store_scatter(posv, [ev], jnp.full((L,), pos_e), mask=lane0)
            plsc.store_scatter(negv, [ev], jnp.full((L,), neg_e), mask=lane0)

        def block(k, carry):
            s = pl.ds(k * L, L)
            su = uv[s]
            si = iv[s]
            sj = jv[s]
            ru = [su[t] for t in range(L)]
            ri = [si[t] for t in range(L)]
            rj = [sj[t] for t in range(L)]
            for t in range(NBUF - 1):
                fire(ru[t], ri[t], rj[t], t)
            for t in range(L):
                if t + NBUF - 1 < L:
                    fire(ru[t + NBUF - 1], ri[t + NBUF - 1],
                         rj[t + NBUF - 1], (t + NBUF - 1) % NBUF)
                wait(t % NBUF)
                compute(k * L + t, ru[t], ri[t], rj[t], t % NBUF)
            return carry

        lax.fori_loop(0, BPW // L, block, 0)

        pltpu.sync_copy(posv, pos_hbm.at[sl])
        pltpu.sync_copy(negv, neg_hbm.at[sl])

    def run(u, i, j, user_table, item_table):
        return bpr(u, i, j, user_table.T, item_table.T)

    return run


def kernel(u, i, j, user_table, item_table):
    B = u.shape[0]
    D = user_table.shape[1]
    run = _build(B, D, user_table.shape[0], item_table.shape[0])
    return run(u, i, j, user_table, item_table)
